# speedup vs baseline: 2.9763x; 2.9763x over previous
import jax
import jax.numpy as jnp
from jax import lax
from jax.experimental import pallas as pl
from jax.experimental.pallas import tpu as pltpu
from jax.experimental.pallas import tpu_sc as plsc

VOCAB = 100000
EMB = 128

_info = plsc.get_sparse_core_info()
NC, NS = _info.num_cores, _info.num_subcores
NW = NC * NS

B = 4096 * 50
B_PER_W = B // NW
CH = 128
N_CHUNKS = B_PER_W // CH


def _body(x_hbm, table_hbm, out_hbm, idx_v, rows_v, gsem):
    wid = lax.axis_index("s") * NC + lax.axis_index("c")
    base = wid * B_PER_W
    pltpu.sync_copy(x_hbm.at[wid], idx_v)

    def step(i, carry):
        pltpu.async_copy(table_hbm.at[idx_v.at[i]], rows_v, gsem).wait()
        pltpu.sync_copy(rows_v, out_hbm.at[pl.ds(base + i * CH, CH)])
        return carry

    lax.fori_loop(0, N_CHUNKS, step, 0)


@jax.jit
def _lookup(x_flat, table):
    mesh = plsc.VectorSubcoreMesh(core_axis_name="c", subcore_axis_name="s")
    return pl.kernel(
        _body,
        out_type=jax.ShapeDtypeStruct((B, EMB), jnp.float32),
        mesh=mesh,
        scratch_types=[
            pltpu.VMEM((N_CHUNKS, CH), jnp.int32),
            pltpu.VMEM((CH, EMB), jnp.float32),
            pltpu.SemaphoreType.DMA,
        ],
    )(x_flat, table)


def kernel(x, table):
    orig_shape = x.shape
    x_flat = x.reshape(NW, N_CHUNKS, CH).astype(jnp.int32)
    out = _lookup(x_flat, table)
    return out.reshape(*orig_shape, EMB)


# speedup vs baseline: 3.3052x; 1.1105x over previous
import jax
import jax.numpy as jnp
from jax import lax
from jax.experimental import pallas as pl
from jax.experimental.pallas import tpu as pltpu
from jax.experimental.pallas import tpu_sc as plsc

VOCAB = 100000
EMB = 128

_info = plsc.get_sparse_core_info()
NC, NS = _info.num_cores, _info.num_subcores
NW = NC * NS

B = 4096 * 50
B_PER_W = B // NW
CH = 128
N_CHUNKS = B_PER_W // CH
NBUF = 5
NG = N_CHUNKS // NBUF


def _body(x_hbm, table_hbm, out_hbm, idx_v, *rest):
    rows = rest[:NBUF]
    gsems = rest[NBUF:2 * NBUF]
    ssems = rest[2 * NBUF:3 * NBUF]
    wid = lax.axis_index("s") * NC + lax.axis_index("c")
    base = wid * B_PER_W
    pltpu.sync_copy(x_hbm.at[wid], idx_v)

    def gather_wait(b):
        pltpu.make_async_copy(table_hbm.at[pl.ds(0, CH)], rows[b],
                              gsems[b]).wait()

    def store_wait(b):
        pltpu.make_async_copy(rows[b], out_hbm.at[pl.ds(0, CH)],
                              ssems[b]).wait()

    for b in range(NBUF):
        pltpu.async_copy(table_hbm.at[idx_v.at[b]], rows[b], gsems[b])

    def grp(t, carry):
        for b in range(NBUF):
            i = t * NBUF + b
            gather_wait(b)
            pltpu.async_copy(rows[b], out_hbm.at[pl.ds(base + i * CH, CH)],
                             ssems[b])

        @pl.when(t < NG - 1)
        def _prefetch():
            for b in range(NBUF):
                store_wait(b)
                pltpu.async_copy(table_hbm.at[idx_v.at[(t + 1) * NBUF + b]],
                                 rows[b], gsems[b])

        return carry

    lax.fori_loop(0, NG, grp, 0)
    for b in range(NBUF):
        store_wait(b)


@jax.jit
def _lookup(x_flat, table):
    mesh = plsc.VectorSubcoreMesh(core_axis_name="c", subcore_axis_name="s")
    return pl.kernel(
        _body,
        out_type=jax.ShapeDtypeStruct((B, EMB), jnp.float32),
        mesh=mesh,
        scratch_types=(
            [pltpu.VMEM((N_CHUNKS, CH), jnp.int32)]
            + [pltpu.VMEM((CH, EMB), jnp.float32) for _ in range(NBUF)]
            + [pltpu.SemaphoreType.DMA for _ in range(2 * NBUF)]
        ),
    )(x_flat, table)


def kernel(x, table):
    orig_shape = x.shape
    x_flat = x.reshape(NW, N_CHUNKS, CH).astype(jnp.int32)
    out = _lookup(x_flat, table)
    return out.reshape(*orig_shape, EMB)
